# baseline (speedup 1.0000x reference)
import jax
import jax.numpy as jnp
from jax import lax
from jax.experimental import pallas as pl
from jax.experimental.pallas import tpu as pltpu
from jax.experimental.pallas import tpu_sc as plsc

B, L, H = 16, 4096, 768
MAX_LABEL = 24
ROWS = 32
NB_SC = 4
NB_TC = B - NB_SC
NC, NS = 2, 16
NW = NC * NS
WPB = NW // NB_SC
TOK_W = (NB_SC * L) // NW
SC_CH = 64
NCH = TOK_W // SC_CH
TC_CH = 2048
TC_NCH = L // TC_CH


def _sc_body(hs_hbm, lm_hbm, z_hbm, sums_hbm, cnts_hbm,
             acc_v, cacc_v, lm_v, rows_v, sem):
    c = lax.axis_index("c")
    s = lax.axis_index("s")
    wid = c * NS + s
    base = NB_TC * L + wid * TOK_W

    pltpu.sync_copy(z_hbm, acc_v)
    zero16 = jnp.zeros((16,), jnp.float32)
    cacc_v[pl.ds(0, 16)] = zero16
    cacc_v[pl.ds(16, 16)] = zero16
    pltpu.sync_copy(lm_hbm.at[pl.ds(base, TOK_W)], lm_v.at[pl.ds(0, TOK_W)])

    ones16 = jnp.ones((16,), jnp.float32)

    def start_gather(i, buf):
        pltpu.async_copy(hs_hbm.at[pl.ds(base + i * SC_CH, SC_CH)],
                         rows_v.at[buf], sem)

    def drain(buf):
        pltpu.make_async_copy(hs_hbm.at[pl.ds(0, SC_CH)],
                              rows_v.at[buf], sem).wait()

    start_gather(0, 0)
    start_gather(1, 1)

    @pl.loop(0, NCH, step=2)
    def _chunks(i):
        for b in range(2):
            drain(b)
            chunk0 = i * SC_CH + b * SC_CH
            for g in range(SC_CH // 16):
                lab16 = lm_v[pl.ds(chunk0 + g * 16, 16)]
                plsc.addupdate_scatter(cacc_v, [lab16], ones16)

            @pl.loop(0, SC_CH)
            def _tok(t):
                rbase = lm_v[pl.ds(chunk0 + t, 16)][0] * H

                @plsc.parallel_loop(0, H, step=16, unroll=8)
                def _col(j):
                    vals = rows_v[b, t, pl.ds(j, 16)]
                    plsc.addupdate(acc_v.at[pl.ds(rbase + j, 16)], vals)

            @pl.when(i + b + 2 < NCH)
            def _():
                start_gather(i + b + 2, b)

    plsc.subcore_barrier()
    pltpu.sync_copy(acc_v, sums_hbm.at[wid])
    pltpu.sync_copy(cacc_v, cnts_hbm.at[wid])


def _sc_segment_sums(hs_flat, lm_flat):
    mesh = plsc.VectorSubcoreMesh(core_axis_name="c", subcore_axis_name="s")
    zeros_acc = jnp.zeros((ROWS * H,), jnp.float32)
    f = pl.kernel(
        _sc_body,
        out_type=[
            jax.ShapeDtypeStruct((NW, ROWS * H), jnp.float32),
            jax.ShapeDtypeStruct((NW, ROWS), jnp.float32),
        ],
        mesh=mesh,
        compiler_params=pltpu.CompilerParams(needs_layout_passes=False),
        scratch_types=[
            pltpu.VMEM((ROWS * H,), jnp.float32),
            pltpu.VMEM((ROWS,), jnp.float32),
            pltpu.VMEM((TOK_W + 16,), jnp.int32),
            pltpu.VMEM((2, SC_CH, H), jnp.float32),
            pltpu.SemaphoreType.DMA,
        ],
    )
    return f(hs_flat, lm_flat, zeros_acc)


def _tc_body(lm_ref, hs_ref, sums_ref, counts_ref):
    c = pl.program_id(1)
    labels = lm_ref[0, 0]
    rows = jax.lax.broadcasted_iota(jnp.int32, (ROWS, TC_CH), 0)
    oh = (labels == rows).astype(jnp.float32)
    contrib = jax.lax.dot(oh, hs_ref[0],
                          precision=jax.lax.Precision.DEFAULT,
                          preferred_element_type=jnp.float32)
    cnt = jnp.broadcast_to(jnp.sum(oh, axis=1)[:, None], (ROWS, 128))

    @pl.when(c == 0)
    def _():
        sums_ref[0] = contrib
        counts_ref[0] = cnt

    @pl.when(c != 0)
    def _():
        sums_ref[0] += contrib
        counts_ref[0] += cnt


def _tc_segment_sums(hidden_tc, lmask_tc):
    lm3 = lmask_tc.astype(jnp.int32).reshape(NB_TC, TC_NCH, 1, TC_CH)
    return pl.pallas_call(
        _tc_body,
        grid=(NB_TC, TC_NCH),
        in_specs=[
            pl.BlockSpec((1, 1, 1, TC_CH), lambda b, c: (b, c, 0, 0)),
            pl.BlockSpec((1, TC_CH, H), lambda b, c: (b, c, 0)),
        ],
        out_specs=[
            pl.BlockSpec((1, ROWS, H), lambda b, c: (b, 0, 0)),
            pl.BlockSpec((1, ROWS, 128), lambda b, c: (b, 0, 0)),
        ],
        out_shape=[
            jax.ShapeDtypeStruct((NB_TC, ROWS, H), jnp.float32),
            jax.ShapeDtypeStruct((NB_TC, ROWS, 128), jnp.float32),
        ],
    )(lm3, hidden_tc)


def _fin_body(ptc_ref, ctc_ref, psc_ref, csc_ref, w_ref, b_ref,
              out_ref, valid_ref):
    psc = jnp.sum(psc_ref[...], axis=1)
    psc = psc.reshape(NB_SC * ROWS, H)
    csc = jnp.sum(csc_ref[...], axis=1, keepdims=True)
    sums = jnp.concatenate([ptc_ref[...], psc], axis=0)
    cnt = jnp.concatenate([ctc_ref[...][:, 0:1], csc], axis=0)
    valid = (cnt > 0).astype(jnp.float32)
    mean = sums / jnp.maximum(cnt, 1.0)
    proj = jax.lax.dot(mean, w_ref[...],
                       precision=jax.lax.Precision.DEFAULT,
                       preferred_element_type=jnp.float32) + b_ref[...]
    out_ref[...] = proj * valid
    valid_ref[...] = jnp.broadcast_to(valid, (B * ROWS, 128))


def _finish(ptc, ctc, psc, csc, W_label, b_label):
    return pl.pallas_call(
        _fin_body,
        out_shape=[
            jax.ShapeDtypeStruct((B * ROWS, H), jnp.float32),
            jax.ShapeDtypeStruct((B * ROWS, 128), jnp.float32),
        ],
    )(ptc, ctc, psc, csc, W_label, b_label.reshape(1, H))


def kernel(hidden_states, lmask, input_ids, attention_mask, W_label, b_label):
    hs_flat = hidden_states.reshape(B * L, H)
    lm_flat = lmask.astype(jnp.int32).reshape(B * L)
    sc_sums, sc_cnts = _sc_segment_sums(hs_flat, lm_flat)
    tc_sums, tc_cnts = _tc_segment_sums(hidden_states[:NB_TC], lmask[:NB_TC])
    out, valid = _finish(tc_sums.reshape(NB_TC * ROWS, H),
                         tc_cnts.reshape(NB_TC * ROWS, 128),
                         sc_sums.reshape(NB_SC, WPB, ROWS, H),
                         sc_cnts.reshape(NB_SC, WPB, ROWS)
                                .transpose(0, 2, 1).reshape(NB_SC * ROWS, WPB),
                         W_label, b_label)
    out3 = out.reshape(B, ROWS, H)[:, 1:MAX_LABEL + 1, :]
    aggregated = out3.reshape(B * MAX_LABEL, H)
    valid_mask = (valid.reshape(B, ROWS, 128)[:, 1:MAX_LABEL + 1, 0] > 0
                  ).reshape(B * MAX_LABEL)
    all_batch_ids = jnp.repeat(jnp.arange(B), MAX_LABEL)
    all_label_ids = jnp.tile(jnp.arange(1, MAX_LABEL + 1), B)
    return aggregated, all_batch_ids, all_label_ids, valid_mask


# speedup vs baseline: 2.6601x; 2.6601x over previous
import functools

import jax
import jax.numpy as jnp
from jax.experimental import pallas as pl
from jax.experimental.pallas import tpu as pltpu

B, L, H = 16, 4096, 768
MAX_LABEL = 24
ROWS = 32
CH = 2048
NCH = L // CH


def _seg_body(lm_ref, hs_ref, sums_ref, counts_ref):
    c = pl.program_id(1)
    labels = lm_ref[0, 0]
    rows = jax.lax.broadcasted_iota(jnp.int32, (ROWS, CH), 0)
    oh = (labels == rows).astype(jnp.float32)
    contrib = jax.lax.dot(oh, hs_ref[0],
                          precision=jax.lax.Precision.DEFAULT,
                          preferred_element_type=jnp.float32)
    cnt = jnp.broadcast_to(jnp.sum(oh, axis=1)[:, None], (ROWS, 128))

    @pl.when(c == 0)
    def _():
        sums_ref[0] = contrib
        counts_ref[0] = cnt

    @pl.when(c != 0)
    def _():
        sums_ref[0] += contrib
        counts_ref[0] += cnt


def _fin_body(sums_ref, counts_ref, w_ref, b_ref, out_ref, valid_ref):
    cnt = counts_ref[:, 0:1]
    valid = (cnt > 0).astype(jnp.float32)
    mean = sums_ref[...] / jnp.maximum(cnt, 1.0)
    proj = jax.lax.dot(mean, w_ref[...],
                       precision=jax.lax.Precision.DEFAULT,
                       preferred_element_type=jnp.float32) + b_ref[...]
    out_ref[...] = proj * valid
    valid_ref[...] = jnp.broadcast_to(valid, (B * ROWS, 128))


@functools.partial(jax.jit, static_argnames=())
def _segment_sums(hidden_states, lmask):
    lm3 = lmask.astype(jnp.int32).reshape(B, NCH, 1, CH)
    sums, counts = pl.pallas_call(
        _seg_body,
        grid=(B, NCH),
        in_specs=[
            pl.BlockSpec((1, 1, 1, CH), lambda b, c: (b, c, 0, 0)),
            pl.BlockSpec((1, CH, H), lambda b, c: (b, c, 0)),
        ],
        out_specs=[
            pl.BlockSpec((1, ROWS, H), lambda b, c: (b, 0, 0)),
            pl.BlockSpec((1, ROWS, 128), lambda b, c: (b, 0, 0)),
        ],
        out_shape=[
            jax.ShapeDtypeStruct((B, ROWS, H), jnp.float32),
            jax.ShapeDtypeStruct((B, ROWS, 128), jnp.float32),
        ],
    )(lm3, hidden_states)
    return sums, counts


def _finish(sums, counts, W_label, b_label):
    out, valid = pl.pallas_call(
        _fin_body,
        out_shape=[
            jax.ShapeDtypeStruct((B * ROWS, H), jnp.float32),
            jax.ShapeDtypeStruct((B * ROWS, 128), jnp.float32),
        ],
    )(sums.reshape(B * ROWS, H), counts.reshape(B * ROWS, 128),
      W_label, b_label.reshape(1, H))
    return out, valid


def kernel(hidden_states, lmask, input_ids, attention_mask, W_label, b_label):
    sums, counts = _segment_sums(hidden_states, lmask)
    out, valid = _finish(sums, counts, W_label, b_label)
    out3 = out.reshape(B, ROWS, H)[:, 1:MAX_LABEL + 1, :]
    aggregated = out3.reshape(B * MAX_LABEL, H)
    valid_mask = (valid.reshape(B, ROWS, 128)[:, 1:MAX_LABEL + 1, 0] > 0
                  ).reshape(B * MAX_LABEL)
    all_batch_ids = jnp.repeat(jnp.arange(B), MAX_LABEL)
    all_label_ids = jnp.tile(jnp.arange(1, MAX_LABEL + 1), B)
    return aggregated, all_batch_ids, all_label_ids, valid_mask
